# baseline (speedup 1.0000x reference)
import functools

import jax
import jax.numpy as jnp
from jax import lax
from jax.experimental import pallas as pl
from jax.experimental.pallas import tpu as pltpu
from jax.experimental.pallas import tpu_sc as plsc

N_PRED = 8
NC = 2
NS = 16
NW = NC * NS
LANES = 16
CHUNK = 1024
N_CHUNKS = 49
PREP_ROWS = 782


def _finish_body(acc_ref, out_ref):
    out_ref[...] = acc_ref[0] + acc_ref[1]


def _sc_edge_body(unary_hbm, b_hbm, p_hbm, q_hbm, bcw_hbm, uw_hbm,
                  acc_out, db_out, ea_hbm, ec_hbm,
                  idxp, idxq, bbuf, eabuf, ecbuf, d1b, d2b, dbo,
                  wbuf, uwbuf, acc, gA, gB, ssem, dsem):
    n_rows_pad = acc.shape[0]
    edges_per_tile = N_CHUNKS * CHUNK
    rows_per_tile = n_rows_pad // NS

    c = lax.axis_index("c")
    s = lax.axis_index("s")
    wid = s * NC + c
    ebase = wid * edges_per_tile

    pltpu.sync_copy(bcw_hbm, wbuf)
    pltpu.sync_copy(uw_hbm, uwbuf)

    iota16 = lax.iota(jnp.int32, LANES)
    io3 = iota16 >> 3
    io7 = iota16 & 7
    io7x = io7 ^ 1
    evenf = (1 - (iota16 & 1)).astype(jnp.float32)
    zero16 = jnp.zeros((LANES,), jnp.float32)

    def zero_body(z, carry):
        rows = z * 2 + io3
        plsc.store_scatter(d1b[1], [rows, io7], zero16)
        return carry

    lax.fori_loop(0, CHUNK // 2, zero_body, 0)

    uw16 = uwbuf[...]

    def prep_sub(k, carry):
        base_r = s * rows_per_tile + k * PREP_ROWS
        pltpu.sync_copy(unary_hbm.at[pl.ds(base_r, PREP_ROWS)],
                        eabuf[0].at[pl.ds(0, PREP_ROWS)])

        def g_body(g, gc):
            rows = g * 2 + io3
            x = plsc.load_gather(eabuf[0], [rows, io7])
            xsw = plsc.load_gather(eabuf[0], [rows, io7x])
            t = 1.0 / (1.0 + jnp.exp(-(x + xsw)))
            u = x + uw16 * (t - evenf)
            ea = jnp.exp(-u)
            plsc.store_scatter(d1b[0], [rows, io7], u)
            plsc.store_scatter(ecbuf[0], [rows, io7], ea)
            plsc.store_scatter(d2b[0], [rows, io7], 1.0 / ea)
            return gc

        lax.fori_loop(0, PREP_ROWS * N_PRED // LANES, g_body, 0)
        pltpu.sync_copy(ecbuf[0].at[pl.ds(0, PREP_ROWS)],
                        ea_hbm.at[pl.ds(base_r, PREP_ROWS)])
        pltpu.sync_copy(d2b[0].at[pl.ds(0, PREP_ROWS)],
                        ec_hbm.at[pl.ds(base_r, PREP_ROWS)])

        @pl.when(c == 0)
        def _():
            pltpu.sync_copy(d1b[0].at[pl.ds(0, PREP_ROWS)],
                            acc.at[pl.ds(base_r, PREP_ROWS)])

        @pl.when(c != 0)
        def _():
            pltpu.sync_copy(d1b[1].at[pl.ds(0, PREP_ROWS)],
                            acc.at[pl.ds(base_r, PREP_ROWS)])

        return carry

    lax.fori_loop(0, rows_per_tile // PREP_ROWS, prep_sub, 0)
    plsc.subcore_barrier()

    wv = [wbuf[i] for i in range(N_PRED)]

    def issue(cn, i4, i2):
        pltpu.sync_copy(p_hbm.at[pl.ds(ebase + cn * CHUNK, CHUNK)], idxp[i4])
        pltpu.sync_copy(q_hbm.at[pl.ds(ebase + cn * CHUNK, CHUNK)], idxq[i4])
        pltpu.sync_copy(b_hbm.at[pl.ds(ebase + cn * CHUNK, CHUNK)], bbuf[i2])
        pltpu.async_copy(ea_hbm.at[idxp[i4]], eabuf[i2], gA[i2])
        pltpu.async_copy(ec_hbm.at[idxq[i4]], ecbuf[i2], gB[i2])

    def drain_gathers(i4, i2):
        pltpu.make_async_copy(ea_hbm.at[idxp[i4]], eabuf[i2], gA[i2]).wait()
        pltpu.make_async_copy(ec_hbm.at[idxq[i4]], ecbuf[i2], gB[i2]).wait()

    def fire_scatters(i4, i2):
        pltpu.async_copy(d1b[i2], acc.at[idxp[i4]], ssem[i2], add=True)
        pltpu.async_copy(d2b[i2], acc.at[idxq[i4]], ssem[i2], add=True)

    def drain_scatters(i4, i2):
        pltpu.make_async_copy(d1b[i2], acc.at[idxp[i4]], ssem[i2]).wait()
        pltpu.make_async_copy(d2b[i2], acc.at[idxq[i4]], ssem[i2]).wait()

    def fire_dbo(cn, i2):
        pltpu.async_copy(dbo[i2], db_out.at[pl.ds(ebase + cn * CHUNK, CHUNK)],
                         dsem[i2])

    def drain_dbo(cn, i2):
        pltpu.make_async_copy(dbo[i2],
                              db_out.at[pl.ds(ebase + cn * CHUNK, CHUNK)],
                              dsem[i2]).wait()

    def compute(i2):
        def r_body(r, rc):
            rows = r * LANES + iota16
            b16 = bbuf[i2][pl.ds(r * LANES, LANES)]
            g = jnp.exp(-b16)
            sacc = jnp.zeros((LANES,), jnp.float32)
            for i in range(N_PRED):
                ci = jnp.full((LANES,), i, jnp.int32)
                a = plsc.load_gather(eabuf[i2], [rows, ci])
                cc = plsc.load_gather(ecbuf[i2], [rows, ci])
                rz = 1.0 / (a + g + cc)
                u = rz * wv[i]
                plsc.store_scatter(d1b[i2], [rows, ci], a * (-u))
                plsc.store_scatter(d2b[i2], [rows, ci], cc * u)
                sacc = sacc + u
            dbo[i2][pl.ds(r * LANES, LANES)] = b16 - g * sacc
            return rc

        lax.fori_loop(0, CHUNK // LANES, r_body, 0)

    issue(0, 0, 0)

    def loop_body(j, carry):
        for t in range(4):
            cn = 4 * j + t
            i2, i4 = t % 2, t
            drain_gathers(i4, i2)
            if t >= 2:
                drain_scatters((t + 2) % 4, i2)
                drain_dbo(cn - 2, i2)
            else:
                @pl.when(j > 0)
                def _():
                    drain_scatters((t + 2) % 4, i2)
                    drain_dbo(cn - 2, i2)
            issue(cn + 1, (t + 1) % 4, (t + 1) % 2)
            compute(i2)
            fire_scatters(i4, i2)
            fire_dbo(cn, i2)
        return carry

    lax.fori_loop(0, (N_CHUNKS - 1) // 4, loop_body, 0)

    cn = N_CHUNKS - 1
    drain_gathers(0, 0)
    drain_scatters(2, 0)
    drain_dbo(cn - 2, 0)
    compute(0)
    fire_scatters(0, 0)
    fire_dbo(cn, 0)
    drain_scatters(3, 1)
    drain_dbo(cn - 1, 1)
    drain_scatters(0, 0)
    drain_dbo(cn, 0)

    plsc.subcore_barrier()
    pltpu.sync_copy(acc.at[pl.ds(s * rows_per_tile, rows_per_tile)],
                    acc_out.at[c, pl.ds(s * rows_per_tile, rows_per_tile)])


def kernel(unary, binary, unary_clause_weights, binary_clause_weights,
           edge_index):
    n_nodes, n_pred = unary.shape
    n_edges = edge_index.shape[1]
    n_rows_pad = ((n_nodes + NS * 8 - 1) // (NS * 8)) * (NS * 8)

    unary_pad = jnp.concatenate(
        [unary, jnp.zeros((n_rows_pad - n_nodes, n_pred), jnp.float32)])
    e_pad = NW * N_CHUNKS * CHUNK
    npad = e_pad - n_edges
    p_arr = jnp.concatenate(
        [edge_index[0], jnp.full((npad,), n_rows_pad - 1, jnp.int32)])
    q_arr = jnp.concatenate(
        [edge_index[1], jnp.full((npad,), n_rows_pad - 1, jnp.int32)])
    b_flat = jnp.concatenate(
        [binary.reshape(n_edges), jnp.zeros((npad,), jnp.float32)])
    wexp = jnp.broadcast_to(binary_clause_weights[:, None], (N_PRED, LANES))
    uw16 = jnp.tile(jnp.repeat(unary_clause_weights, 2), 2)

    sc_edge = functools.partial(
        pl.kernel,
        out_type=[
            jax.ShapeDtypeStruct((NC, n_rows_pad, n_pred), jnp.float32),
            jax.ShapeDtypeStruct((e_pad,), jnp.float32),
            jax.ShapeDtypeStruct((n_rows_pad, n_pred), jnp.float32),
            jax.ShapeDtypeStruct((n_rows_pad, n_pred), jnp.float32),
        ],
        mesh=plsc.VectorSubcoreMesh(core_axis_name="c", subcore_axis_name="s"),
        compiler_params=pltpu.CompilerParams(needs_layout_passes=False,
                                             use_tc_tiling_on_sc=False),
        scratch_types=[
            [pltpu.VMEM((CHUNK,), jnp.int32) for _ in range(4)],
            [pltpu.VMEM((CHUNK,), jnp.int32) for _ in range(4)],
            [pltpu.VMEM((CHUNK,), jnp.float32) for _ in range(2)],
            [pltpu.VMEM((CHUNK, N_PRED), jnp.float32) for _ in range(2)],
            [pltpu.VMEM((CHUNK, N_PRED), jnp.float32) for _ in range(2)],
            [pltpu.VMEM((CHUNK, N_PRED), jnp.float32) for _ in range(2)],
            [pltpu.VMEM((CHUNK, N_PRED), jnp.float32) for _ in range(2)],
            [pltpu.VMEM((CHUNK,), jnp.float32) for _ in range(2)],
            pltpu.VMEM((N_PRED, LANES), jnp.float32),
            pltpu.VMEM((LANES,), jnp.float32),
            pltpu.VMEM_SHARED((n_rows_pad, n_pred), jnp.float32),
            [pltpu.SemaphoreType.DMA for _ in range(2)],
            [pltpu.SemaphoreType.DMA for _ in range(2)],
            [pltpu.SemaphoreType.DMA for _ in range(2)],
            [pltpu.SemaphoreType.DMA for _ in range(2)],
        ],
    )(_sc_edge_body)
    acc_parts, db, _ea, _ec = sc_edge(unary_pad, b_flat, p_arr, q_arr,
                                      wexp, uw16)

    row_blk = 4000
    u_out = pl.pallas_call(
        _finish_body,
        grid=(n_nodes // row_blk,),
        in_specs=[pl.BlockSpec((NC, row_blk, n_pred), lambda i: (0, i, 0))],
        out_specs=pl.BlockSpec((row_blk, n_pred), lambda i: (i, 0)),
        out_shape=jax.ShapeDtypeStruct((n_nodes, n_pred), jnp.float32),
    )(acc_parts)

    return (u_out, db[:n_edges].reshape(n_edges, 1))


# speedup vs baseline: 1.2360x; 1.2360x over previous
import functools

import jax
import jax.numpy as jnp
from jax import lax
from jax.experimental import pallas as pl
from jax.experimental.pallas import tpu as pltpu
from jax.experimental.pallas import tpu_sc as plsc

N_PRED = 8
NC = 2
NS = 16
NW = NC * NS
LANES = 16
SUB = 128
N_SUB = 8
CHUNK = SUB * N_SUB
N_CHUNKS = 49


def _prep_body(x_ref, w_ref, u_ref, ea_ref, ec_ref):
    x = x_ref[...]
    r, cols = x.shape
    ci = lax.broadcasted_iota(jnp.int32, (cols, cols), 1)
    ri = lax.broadcasted_iota(jnp.int32, (cols, cols), 0)
    pair = ((ci // 2 == ri // 2) & (ci // N_PRED == ri // N_PRED))
    s = jnp.dot(x, pair.astype(jnp.float32), preferred_element_type=jnp.float32)
    t = 1.0 / (1.0 + jnp.exp(-s))
    even = (lax.broadcasted_iota(jnp.int32, (1, cols), 1) % 2 == 0)
    u = x + w_ref[...] * (t - even.astype(jnp.float32))
    pad = u_ref.shape[0] - r
    zrows = jnp.zeros((pad, cols), jnp.float32)
    ea = jnp.exp(-u)
    u_ref[...] = jnp.concatenate([u, zrows], axis=0)
    ea_ref[...] = jnp.concatenate([ea, zrows], axis=0)
    ec_ref[...] = jnp.concatenate([1.0 / ea, zrows], axis=0)


def _finish_body(u_ref, acc_ref, out_ref):
    out_ref[...] = u_ref[...] + acc_ref[0] + acc_ref[1]


def _sc_edge_body(ea_hbm, ec_hbm, b_hbm, p_hbm, q_hbm, bcw_hbm, z_hbm,
                  acc_out, db_out,
                  idxp, idxq, bbuf, eabuf, ecbuf, d1b, d2b, dbo, wbuf,
                  acc, gA, gB, ssem, dsem):
    n_rows_pad = acc.shape[0]
    edges_per_tile = N_CHUNKS * CHUNK
    rows_per_tile = n_rows_pad // NS

    c = lax.axis_index("c")
    s = lax.axis_index("s")
    wid = s * NC + c
    ebase = wid * edges_per_tile
    rbase = wid * (edges_per_tile // SUB)

    pltpu.sync_copy(bcw_hbm, wbuf)
    pltpu.sync_copy(z_hbm.at[pl.ds(s * rows_per_tile, rows_per_tile)],
                    acc.at[pl.ds(s * rows_per_tile, rows_per_tile)])
    plsc.subcore_barrier()

    iota16 = lax.iota(jnp.int32, LANES)
    wv = [wbuf[i] for i in range(N_PRED)]

    def issue(cn, i4, i2):
        row = rbase + cn * N_SUB
        pltpu.sync_copy(p_hbm.at[pl.ds(row, N_SUB)], idxp[i4])
        pltpu.sync_copy(q_hbm.at[pl.ds(row, N_SUB)], idxq[i4])
        pltpu.sync_copy(b_hbm.at[pl.ds(ebase + cn * CHUNK, CHUNK)], bbuf[i2])
        for k in range(N_SUB):
            pltpu.async_copy(ea_hbm.at[idxp[i4].at[k]],
                             eabuf[i2].at[pl.ds(k * SUB, SUB)], gA[i2])
            pltpu.async_copy(ec_hbm.at[idxq[i4].at[k]],
                             ecbuf[i2].at[pl.ds(k * SUB, SUB)], gB[i2])

    def drain_gathers(i4, i2):
        for k in range(N_SUB):
            pltpu.make_async_copy(ea_hbm.at[idxp[i4].at[k]],
                                  eabuf[i2].at[pl.ds(k * SUB, SUB)],
                                  gA[i2]).wait()
            pltpu.make_async_copy(ec_hbm.at[idxq[i4].at[k]],
                                  ecbuf[i2].at[pl.ds(k * SUB, SUB)],
                                  gB[i2]).wait()

    def fire_scatters(i4, i2):
        for k in range(N_SUB):
            pltpu.async_copy(d1b[i2].at[pl.ds(k * SUB, SUB)],
                             acc.at[idxp[i4].at[k]], ssem[i2], add=True)
            pltpu.async_copy(d2b[i2].at[pl.ds(k * SUB, SUB)],
                             acc.at[idxq[i4].at[k]], ssem[i2], add=True)

    def drain_scatters(i4, i2):
        for k in range(N_SUB):
            pltpu.make_async_copy(d1b[i2].at[pl.ds(k * SUB, SUB)],
                                  acc.at[idxp[i4].at[k]], ssem[i2]).wait()
            pltpu.make_async_copy(d2b[i2].at[pl.ds(k * SUB, SUB)],
                                  acc.at[idxq[i4].at[k]], ssem[i2]).wait()

    def fire_dbo(cn, i2):
        pltpu.async_copy(dbo[i2], db_out.at[pl.ds(ebase + cn * CHUNK, CHUNK)],
                         dsem[i2])

    def drain_dbo(cn, i2):
        pltpu.make_async_copy(dbo[i2],
                              db_out.at[pl.ds(ebase + cn * CHUNK, CHUNK)],
                              dsem[i2]).wait()

    def compute(i2):
        def r_body(r, rc):
            rows = r * LANES + iota16
            b16 = bbuf[i2][pl.ds(r * LANES, LANES)]
            g = jnp.exp(-b16)
            sacc = jnp.zeros((LANES,), jnp.float32)
            for i in range(N_PRED):
                ci = jnp.full((LANES,), i, jnp.int32)
                a = plsc.load_gather(eabuf[i2], [rows, ci])
                cc = plsc.load_gather(ecbuf[i2], [rows, ci])
                rz = 1.0 / (a + g + cc)
                u = rz * wv[i]
                plsc.store_scatter(d1b[i2], [rows, ci], a * (-u))
                plsc.store_scatter(d2b[i2], [rows, ci], cc * u)
                sacc = sacc + u
            dbo[i2][pl.ds(r * LANES, LANES)] = b16 - g * sacc
            return rc

        lax.fori_loop(0, CHUNK // LANES, r_body, 0)

    issue(0, 0, 0)

    def loop_body(j, carry):
        for t in range(4):
            cn = 4 * j + t
            i2, i4 = t % 2, t
            drain_gathers(i4, i2)
            if t >= 2:
                drain_scatters((t + 2) % 4, i2)
                drain_dbo(cn - 2, i2)
            else:
                @pl.when(j > 0)
                def _():
                    drain_scatters((t + 2) % 4, i2)
                    drain_dbo(cn - 2, i2)
            issue(cn + 1, (t + 1) % 4, (t + 1) % 2)
            compute(i2)
            fire_scatters(i4, i2)
            fire_dbo(cn, i2)
        return carry

    lax.fori_loop(0, (N_CHUNKS - 1) // 4, loop_body, 0)

    cn = N_CHUNKS - 1
    drain_gathers(0, 0)
    drain_scatters(2, 0)
    drain_dbo(cn - 2, 0)
    compute(0)
    fire_scatters(0, 0)
    fire_dbo(cn, 0)
    drain_scatters(3, 1)
    drain_dbo(cn - 1, 1)
    drain_scatters(0, 0)
    drain_dbo(cn, 0)

    plsc.subcore_barrier()
    pltpu.sync_copy(acc.at[pl.ds(s * rows_per_tile, rows_per_tile)],
                    acc_out.at[c, pl.ds(s * rows_per_tile, rows_per_tile)])


def kernel(unary, binary, unary_clause_weights, binary_clause_weights,
           edge_index):
    n_nodes, n_pred = unary.shape
    n_edges = edge_index.shape[1]
    n_rows_pad = ((n_nodes + NS * 8 - 1) // (NS * 8)) * (NS * 8)
    wide = 128
    n_wrows = n_nodes * n_pred // wide
    n_wrows_pad = n_rows_pad * n_pred // wide

    wrow = jnp.tile(jnp.repeat(unary_clause_weights, 2), wide // n_pred)
    u, ea, ec = pl.pallas_call(
        _prep_body,
        in_specs=[
            pl.BlockSpec((n_wrows, wide), lambda: (0, 0)),
            pl.BlockSpec((1, wide), lambda: (0, 0)),
        ],
        out_specs=[pl.BlockSpec((n_wrows_pad, wide), lambda: (0, 0))] * 3,
        out_shape=[jax.ShapeDtypeStruct((n_wrows_pad, wide), jnp.float32)] * 3,
    )(unary.reshape(n_wrows, wide), wrow.reshape(1, wide))

    e_pad = NW * N_CHUNKS * CHUNK
    npad = e_pad - n_edges
    p_arr = jnp.concatenate(
        [edge_index[0], jnp.full((npad,), n_rows_pad - 1, jnp.int32)]
    ).reshape(e_pad // SUB, SUB)
    q_arr = jnp.concatenate(
        [edge_index[1], jnp.full((npad,), n_rows_pad - 1, jnp.int32)]
    ).reshape(e_pad // SUB, SUB)
    b_flat = jnp.concatenate(
        [binary.reshape(n_edges), jnp.zeros((npad,), jnp.float32)])
    zeros = jnp.zeros((n_rows_pad, n_pred), jnp.float32)
    wexp = jnp.broadcast_to(binary_clause_weights[:, None], (N_PRED, LANES))

    sc_edge = functools.partial(
        pl.kernel,
        out_type=[
            jax.ShapeDtypeStruct((NC, n_rows_pad, n_pred), jnp.float32),
            jax.ShapeDtypeStruct((e_pad,), jnp.float32),
        ],
        mesh=plsc.VectorSubcoreMesh(core_axis_name="c", subcore_axis_name="s"),
        compiler_params=pltpu.CompilerParams(needs_layout_passes=False,
                                             use_tc_tiling_on_sc=False),
        scratch_types=[
            [pltpu.VMEM((N_SUB, SUB), jnp.int32) for _ in range(4)],
            [pltpu.VMEM((N_SUB, SUB), jnp.int32) for _ in range(4)],
            [pltpu.VMEM((CHUNK,), jnp.float32) for _ in range(2)],
            [pltpu.VMEM((CHUNK, N_PRED), jnp.float32) for _ in range(2)],
            [pltpu.VMEM((CHUNK, N_PRED), jnp.float32) for _ in range(2)],
            [pltpu.VMEM((CHUNK, N_PRED), jnp.float32) for _ in range(2)],
            [pltpu.VMEM((CHUNK, N_PRED), jnp.float32) for _ in range(2)],
            [pltpu.VMEM((CHUNK,), jnp.float32) for _ in range(2)],
            pltpu.VMEM((N_PRED, LANES), jnp.float32),
            pltpu.VMEM_SHARED((n_rows_pad, n_pred), jnp.float32),
            [pltpu.SemaphoreType.DMA for _ in range(2)],
            [pltpu.SemaphoreType.DMA for _ in range(2)],
            [pltpu.SemaphoreType.DMA for _ in range(2)],
            [pltpu.SemaphoreType.DMA for _ in range(2)],
        ],
    )(_sc_edge_body)
    acc_parts, db = sc_edge(ea.reshape(n_rows_pad, n_pred),
                            ec.reshape(n_rows_pad, n_pred),
                            b_flat, p_arr, q_arr, wexp, zeros)

    u_out = pl.pallas_call(
        _finish_body,
        in_specs=[
            pl.BlockSpec((n_wrows_pad, wide), lambda: (0, 0)),
            pl.BlockSpec((NC, n_wrows_pad, wide), lambda: (0, 0, 0)),
        ],
        out_specs=pl.BlockSpec((n_wrows_pad, wide), lambda: (0, 0)),
        out_shape=jax.ShapeDtypeStruct((n_wrows_pad, wide), jnp.float32),
    )(u, acc_parts.reshape(NC, n_wrows_pad, wide))

    u_final = u_out.reshape(n_rows_pad, n_pred)[:n_nodes]
    return (u_final, db[:n_edges].reshape(n_edges, 1))
